# Initial kernel scaffold; baseline (speedup 1.0000x reference)
import jax
import jax.numpy as jnp
from jax.experimental import pallas as pl


def kernel(node_inputs, src_ids, dst_ids, W1, b1, W2, b2, W3, b3, Wih, bih, Whh, bhh, Wout, bout):
    raise NotImplementedError("write your pallas kernel here")


# speedup vs baseline: 3.3896x; 3.3896x over previous
import functools

import jax
import jax.numpy as jnp
from jax import lax
from jax.experimental import pallas as pl
from jax.experimental.pallas import tpu as pltpu
from jax.experimental.pallas import tpu_sc as plsc

N_NODES = 100000
N_EDGES = 1600000
N_ITERS = 7
NF = 10
NI = 9
EF = 11
NO = 9

HP = 16
NC = 2
NS = 16
NW = NC * NS
E_PER_W = N_EDGES // NW
CHUNK = 2000
N_CHUNK = E_PER_W // CHUNK
SCHUNK = 1000
N_SCHUNK = E_PER_W // SCHUNK
NODES_PER_TILE = N_NODES // NS


def _pad2(a, r, c):
    return jnp.zeros((r, c), jnp.float32).at[: a.shape[0], : a.shape[1]].set(a)


def _sc_gather_body(table, src_hbm, dst_hbm, out_s, out_d,
                    idx_s, idx_d, rows_s, rows_d, sem_s, sem_d):
    wid = lax.axis_index("s") * NC + lax.axis_index("c")
    base0 = wid * E_PER_W

    @pl.loop(0, N_CHUNK)
    def _chunks(j):
        base = base0 + j * CHUNK
        pltpu.sync_copy(src_hbm.at[pl.ds(base, CHUNK)], idx_s)
        pltpu.sync_copy(dst_hbm.at[pl.ds(base, CHUNK)], idx_d)
        cp_s = pltpu.async_copy(table.at[idx_s], rows_s, sem_s)
        cp_d = pltpu.async_copy(table.at[idx_d], rows_d, sem_d)
        cp_s.wait()
        cp_d.wait()
        pltpu.sync_copy(rows_s, out_s.at[pl.ds(base, CHUNK)])
        pltpu.sync_copy(rows_d, out_d.at[pl.ds(base, CHUNK)])


@functools.cache
def _sc_gather_kernel():
    return pl.kernel(
        _sc_gather_body,
        out_type=(
            jax.ShapeDtypeStruct((N_EDGES, HP), jnp.float32),
            jax.ShapeDtypeStruct((N_EDGES, HP), jnp.float32),
        ),
        mesh=plsc.VectorSubcoreMesh(core_axis_name="c", subcore_axis_name="s",
                                    num_cores=NC, num_subcores=NS),
        scratch_types=[
            pltpu.VMEM((CHUNK,), jnp.int32),
            pltpu.VMEM((CHUNK,), jnp.int32),
            pltpu.VMEM((CHUNK, HP), jnp.float32),
            pltpu.VMEM((CHUNK, HP), jnp.float32),
            pltpu.SemaphoreType.DMA,
            pltpu.SemaphoreType.DMA,
        ],
        compiler_params=pltpu.CompilerParams(use_tc_tiling_on_sc=False),
    )


def _sc_gather(hidden, src_ids, dst_ids):
    return _sc_gather_kernel()(hidden, src_ids, dst_ids)


def _sc_scatter_body(msg_hbm, dst_hbm, zeros_hbm, out_part,
                     acc, idx_v, msg_v, sem):
    c = lax.axis_index("c")
    s = lax.axis_index("s")
    wid = s * NC + c
    nbase = s * NODES_PER_TILE
    pltpu.sync_copy(zeros_hbm.at[pl.ds(nbase, NODES_PER_TILE)],
                    acc.at[pl.ds(nbase, NODES_PER_TILE)])
    plsc.subcore_barrier()

    base0 = wid * E_PER_W

    @pl.loop(0, N_SCHUNK)
    def _chunks(j):
        base = base0 + j * SCHUNK
        pltpu.sync_copy(dst_hbm.at[pl.ds(base, SCHUNK)], idx_v)
        pltpu.sync_copy(msg_hbm.at[pl.ds(base, SCHUNK)], msg_v)
        pltpu.sync_copy(msg_v, acc.at[idx_v], add=True)

    plsc.subcore_barrier()
    pltpu.sync_copy(acc.at[pl.ds(nbase, NODES_PER_TILE)],
                    out_part.at[c, pl.ds(nbase, NODES_PER_TILE)])


@functools.cache
def _sc_scatter_kernel():
    return pl.kernel(
        _sc_scatter_body,
        out_type=jax.ShapeDtypeStruct((NC, N_NODES, HP), jnp.float32),
        mesh=plsc.VectorSubcoreMesh(core_axis_name="c", subcore_axis_name="s",
                                    num_cores=NC, num_subcores=NS),
        scratch_types=[
            pltpu.VMEM_SHARED((N_NODES, HP), jnp.float32),
            pltpu.VMEM((SCHUNK,), jnp.int32),
            pltpu.VMEM((SCHUNK, HP), jnp.float32),
            pltpu.SemaphoreType.DMA,
        ],
        compiler_params=pltpu.CompilerParams(use_tc_tiling_on_sc=False),
    )


def _sc_scatter(msg, dst_ids, zeros_nodes):
    return _sc_scatter_kernel()(msg, dst_ids, zeros_nodes)


BE = 3200


def _mlp_body(gs_ref, gd_ref, a1_ref, b1l_ref, b1_ref, w2_ref, b2_ref,
              w3_ref, b3_ref, out_ref):
    gs = gs_ref[...]
    gd = gd_ref[...]
    m1 = jnp.dot(gs, a1_ref[...], preferred_element_type=jnp.float32)
    m1 = m1 + jnp.dot(gd, b1l_ref[...], preferred_element_type=jnp.float32)
    m1 = jnp.maximum(m1 + b1_ref[...], 0.0)
    m2 = jnp.dot(m1, w2_ref[...], preferred_element_type=jnp.float32)
    m2 = jnp.maximum(m2 + b2_ref[...], 0.0)
    m3 = jnp.dot(m2, w3_ref[...], preferred_element_type=jnp.float32)
    out_ref[...] = m3 + b3_ref[...]


def _tc_mlp(gs, gd, a1, b1l, b1, w2, b2, w3, b3):
    grid = (N_EDGES // BE,)
    full = lambda shape: pl.BlockSpec(shape, lambda i: (0, 0))
    return pl.pallas_call(
        _mlp_body,
        grid=grid,
        in_specs=[
            pl.BlockSpec((BE, HP), lambda i: (i, 0)),
            pl.BlockSpec((BE, HP), lambda i: (i, 0)),
            full((HP, 128)), full((HP, 128)), full((1, 128)),
            full((128, 128)), full((1, 128)),
            full((128, HP)), full((1, HP)),
        ],
        out_specs=pl.BlockSpec((BE, HP), lambda i: (i, 0)),
        out_shape=jax.ShapeDtypeStruct((N_EDGES, HP), jnp.float32),
    )(gs, gd, a1, b1l, b1, w2, b2, w3, b3)


BN = 5000


def _gru_body(ni_ref, part_ref, h_ref, wn_ref, wa_ref, wh_ref, bi_ref,
              bh_ref, wo_ref, bo_ref, hout_ref, oout_ref):
    ni = ni_ref[...]
    agg = part_ref[0] + part_ref[1]
    h = h_ref[...]

    def gates(g):
        gx = jnp.dot(ni, wn_ref[g], preferred_element_type=jnp.float32)
        gx = gx + jnp.dot(agg, wa_ref[g], preferred_element_type=jnp.float32)
        gx = gx + bi_ref[g]
        gh = jnp.dot(h, wh_ref[g], preferred_element_type=jnp.float32)
        gh = gh + bh_ref[g]
        return gx, gh

    rx, rh = gates(0)
    zx, zh = gates(1)
    nx, nh = gates(2)
    r = jax.nn.sigmoid(rx + rh)
    z = jax.nn.sigmoid(zx + zh)
    n = jnp.tanh(nx + r * nh)
    hn = (1.0 - z) * n + z * h
    hout_ref[...] = hn
    oout_ref[...] = jnp.dot(hn, wo_ref[...],
                            preferred_element_type=jnp.float32) + bo_ref[...]


def _tc_gru(ni, parts, h, wn, wa, wh, bi, bh, wo, bo):
    grid = (N_NODES // BN,)
    full = lambda shape: pl.BlockSpec(shape, lambda i: tuple(0 for _ in shape))
    return pl.pallas_call(
        _gru_body,
        grid=grid,
        in_specs=[
            pl.BlockSpec((BN, HP), lambda i: (i, 0)),
            pl.BlockSpec((NC, BN, HP), lambda i: (0, i, 0)),
            pl.BlockSpec((BN, HP), lambda i: (i, 0)),
            full((3, HP, HP)), full((3, HP, HP)), full((3, HP, HP)),
            full((3, 1, HP)), full((3, 1, HP)),
            full((HP, HP)), full((1, HP)),
        ],
        out_specs=[
            pl.BlockSpec((BN, HP), lambda i: (i, 0)),
            pl.BlockSpec((BN, HP), lambda i: (i, 0)),
        ],
        out_shape=[
            jax.ShapeDtypeStruct((N_NODES, HP), jnp.float32),
            jax.ShapeDtypeStruct((N_NODES, HP), jnp.float32),
        ],
    )(ni, parts, h, wn, wa, wh, bi, bh, wo, bo)


def kernel(node_inputs, src_ids, dst_ids, W1, b1, W2, b2, W3, b3,
           Wih, bih, Whh, bhh, Wout, bout):
    src_ids = src_ids.astype(jnp.int32)
    dst_ids = dst_ids.astype(jnp.int32)

    a1 = _pad2(W1[:, :NF].T, HP, 128)
    b1l = _pad2(W1[:, NF:].T, HP, 128)
    b1p = _pad2(b1[None, :], 1, 128)
    w2 = _pad2(W2.T, 128, 128)
    b2p = _pad2(b2[None, :], 1, 128)
    w3 = _pad2(W3.T, 128, HP)
    b3p = _pad2(b3[None, :], 1, HP)

    wn = jnp.stack([_pad2(Wih[g * NF:(g + 1) * NF, :NI].T, HP, HP)
                    for g in range(3)])
    wa = jnp.stack([_pad2(Wih[g * NF:(g + 1) * NF, NI:].T, HP, HP)
                    for g in range(3)])
    wh = jnp.stack([_pad2(Whh[g * NF:(g + 1) * NF, :].T, HP, HP)
                    for g in range(3)])
    bi = jnp.stack([_pad2(bih[None, g * NF:(g + 1) * NF], 1, HP)
                    for g in range(3)])
    bh = jnp.stack([_pad2(bhh[None, g * NF:(g + 1) * NF], 1, HP)
                    for g in range(3)])
    wo = _pad2(Wout.T, HP, HP)
    bo = _pad2(bout[None, :], 1, HP)

    ni = _pad2(node_inputs, N_NODES, HP)
    zeros_nodes = jnp.zeros((N_NODES, HP), jnp.float32)

    hidden = jnp.zeros((N_NODES, HP), jnp.float32)
    outs = []
    for _ in range(N_ITERS):
        gs, gd = _sc_gather(hidden, src_ids, dst_ids)
        msg = _tc_mlp(gs, gd, a1, b1l, b1p, w2, b2p, w3, b3p)
        parts = _sc_scatter(msg, dst_ids, zeros_nodes)
        hidden, out_it = _tc_gru(ni, parts, hidden, wn, wa, wh, bi, bh,
                                 wo, bo)
        outs.append(out_it)
    return jnp.stack(outs, axis=0)[:, :, :NO]


# speedup vs baseline: 8.6288x; 2.5457x over previous
import functools

import jax
import jax.numpy as jnp
from jax import lax
from jax.experimental import pallas as pl
from jax.experimental.pallas import tpu as pltpu
from jax.experimental.pallas import tpu_sc as plsc

N_NODES = 100000
N_EDGES = 1600000
N_ITERS = 7
NF = 10
NI = 9
EF = 11
NO = 9

HP = 16
NC = 2
NS = 16
NW = NC * NS
E_PER_W = N_EDGES // NW
CHUNK = 2000
N_CHUNK = E_PER_W // CHUNK
SCHUNK = 1000
N_SCHUNK = E_PER_W // SCHUNK
NODES_PER_TILE = N_NODES // NS


def _pad2(a, r, c):
    return jnp.zeros((r, c), jnp.float32).at[: a.shape[0], : a.shape[1]].set(a)


def _sc_gather_body(table, src_hbm, dst_hbm, out_s, out_d,
                    idx_s, idx_d, rows_s, rows_d, sem_s, sem_d):
    wid = lax.axis_index("s") * NC + lax.axis_index("c")
    base0 = wid * E_PER_W

    @pl.loop(0, N_CHUNK)
    def _chunks(j):
        base = base0 + j * CHUNK
        pltpu.sync_copy(src_hbm.at[pl.ds(base, CHUNK)], idx_s)
        pltpu.sync_copy(dst_hbm.at[pl.ds(base, CHUNK)], idx_d)
        cp_s = pltpu.async_copy(table.at[idx_s], rows_s, sem_s)
        cp_d = pltpu.async_copy(table.at[idx_d], rows_d, sem_d)
        cp_s.wait()
        cp_d.wait()
        pltpu.sync_copy(rows_s, out_s.at[pl.ds(base, CHUNK)])
        pltpu.sync_copy(rows_d, out_d.at[pl.ds(base, CHUNK)])


@functools.cache
def _sc_gather_kernel():
    return pl.kernel(
        _sc_gather_body,
        out_type=(
            jax.ShapeDtypeStruct((N_EDGES, HP), jnp.float32),
            jax.ShapeDtypeStruct((N_EDGES, HP), jnp.float32),
        ),
        mesh=plsc.VectorSubcoreMesh(core_axis_name="c", subcore_axis_name="s",
                                    num_cores=NC, num_subcores=NS),
        scratch_types=[
            pltpu.VMEM((CHUNK,), jnp.int32),
            pltpu.VMEM((CHUNK,), jnp.int32),
            pltpu.VMEM((CHUNK, HP), jnp.float32),
            pltpu.VMEM((CHUNK, HP), jnp.float32),
            pltpu.SemaphoreType.DMA,
            pltpu.SemaphoreType.DMA,
        ],
        compiler_params=pltpu.CompilerParams(use_tc_tiling_on_sc=False),
    )


def _sc_gather(hidden, src_ids, dst_ids):
    return _sc_gather_kernel()(hidden, src_ids, dst_ids)


def _sc_scatter_body(msg_hbm, dst_hbm, zeros_hbm, out_part,
                     acc, idx_v, msg_v, sem):
    c = lax.axis_index("c")
    s = lax.axis_index("s")
    wid = s * NC + c
    nbase = s * NODES_PER_TILE
    pltpu.sync_copy(zeros_hbm.at[pl.ds(nbase, NODES_PER_TILE)],
                    acc.at[pl.ds(nbase, NODES_PER_TILE)])
    plsc.subcore_barrier()

    base0 = wid * E_PER_W

    @pl.loop(0, N_SCHUNK)
    def _chunks(j):
        base = base0 + j * SCHUNK
        pltpu.sync_copy(dst_hbm.at[pl.ds(base, SCHUNK)], idx_v)
        pltpu.sync_copy(msg_hbm.at[pl.ds(base, SCHUNK)], msg_v)
        pltpu.sync_copy(msg_v, acc.at[idx_v], add=True)

    plsc.subcore_barrier()
    pltpu.sync_copy(acc.at[pl.ds(nbase, NODES_PER_TILE)],
                    out_part.at[c, pl.ds(nbase, NODES_PER_TILE)])


@functools.cache
def _sc_scatter_kernel():
    return pl.kernel(
        _sc_scatter_body,
        out_type=jax.ShapeDtypeStruct((NC, N_NODES, HP), jnp.float32),
        mesh=plsc.VectorSubcoreMesh(core_axis_name="c", subcore_axis_name="s",
                                    num_cores=NC, num_subcores=NS),
        scratch_types=[
            pltpu.VMEM_SHARED((N_NODES, HP), jnp.float32),
            pltpu.VMEM((SCHUNK,), jnp.int32),
            pltpu.VMEM((SCHUNK, HP), jnp.float32),
            pltpu.SemaphoreType.DMA,
        ],
        compiler_params=pltpu.CompilerParams(use_tc_tiling_on_sc=False),
    )


def _sc_scatter(msg, dst_ids, zeros_nodes):
    return _sc_scatter_kernel()(msg, dst_ids, zeros_nodes)


PACK = 128 // HP
E_ROWS = N_EDGES // PACK
N_ROWS = N_NODES // PACK
BE = 800


def _mlp_body(gs_ref, gd_ref, a1_ref, b1l_ref, b1_ref, w2_ref, b2_ref,
              w3_ref, b3_ref, out_ref):
    gs = gs_ref[...]
    gd = gd_ref[...]
    pieces = []
    for j in range(PACK):
        xs = gs[:, j * HP:(j + 1) * HP]
        xd = gd[:, j * HP:(j + 1) * HP]
        m1 = jnp.dot(xs, a1_ref[...], preferred_element_type=jnp.float32)
        m1 = m1 + jnp.dot(xd, b1l_ref[...], preferred_element_type=jnp.float32)
        m1 = jnp.maximum(m1 + b1_ref[...], 0.0)
        m2 = jnp.dot(m1, w2_ref[...], preferred_element_type=jnp.float32)
        m2 = jnp.maximum(m2 + b2_ref[...], 0.0)
        m3 = jnp.dot(m2, w3_ref[...], preferred_element_type=jnp.float32)
        pieces.append(m3 + b3_ref[...])
    out_ref[...] = jnp.concatenate(pieces, axis=1)


def _tc_mlp(gs, gd, a1, b1l, b1, w2, b2, w3, b3):
    grid = (E_ROWS // BE,)
    full = lambda shape: pl.BlockSpec(shape, lambda i: (0, 0))
    return pl.pallas_call(
        _mlp_body,
        grid=grid,
        in_specs=[
            pl.BlockSpec((BE, 128), lambda i: (i, 0)),
            pl.BlockSpec((BE, 128), lambda i: (i, 0)),
            full((HP, 128)), full((HP, 128)), full((1, 128)),
            full((128, 128)), full((1, 128)),
            full((128, HP)), full((1, HP)),
        ],
        out_specs=pl.BlockSpec((BE, 128), lambda i: (i, 0)),
        out_shape=jax.ShapeDtypeStruct((E_ROWS, 128), jnp.float32),
    )(gs, gd, a1, b1l, b1, w2, b2, w3, b3)


GRU_GRID = 4
BN = N_ROWS // GRU_GRID


def _gru_body(ni_ref, part_ref, h_ref, wn_ref, wa_ref, wh_ref, bi_ref,
              bh_ref, wo_ref, bo_ref, hout_ref, oout_ref):
    ni = ni_ref[0]
    agg = part_ref[0, 0] + part_ref[1, 0]
    h = h_ref[0]
    h_pieces = []
    o_pieces = []
    for j in range(PACK):
        nij = ni[:, j * HP:(j + 1) * HP]
        aj = agg[:, j * HP:(j + 1) * HP]
        hj = h[:, j * HP:(j + 1) * HP]

        def gates(g):
            gx = jnp.dot(nij, wn_ref[g], preferred_element_type=jnp.float32)
            gx = gx + jnp.dot(aj, wa_ref[g],
                              preferred_element_type=jnp.float32)
            gx = gx + bi_ref[g]
            gh = jnp.dot(hj, wh_ref[g], preferred_element_type=jnp.float32)
            gh = gh + bh_ref[g]
            return gx, gh

        rx, rh = gates(0)
        zx, zh = gates(1)
        nx, nh = gates(2)
        r = jax.nn.sigmoid(rx + rh)
        z = jax.nn.sigmoid(zx + zh)
        n = jnp.tanh(nx + r * nh)
        hn = (1.0 - z) * n + z * hj
        h_pieces.append(hn)
        o_pieces.append(jnp.dot(hn, wo_ref[...],
                                preferred_element_type=jnp.float32)
                        + bo_ref[...])
    hout_ref[0] = jnp.concatenate(h_pieces, axis=1)
    oout_ref[0] = jnp.concatenate(o_pieces, axis=1)


def _tc_gru(ni, parts, h, wn, wa, wh, bi, bh, wo, bo):
    ni = ni.reshape(GRU_GRID, BN, 128)
    parts = parts.reshape(NC, GRU_GRID, BN, 128)
    h = h.reshape(GRU_GRID, BN, 128)
    full = lambda shape: pl.BlockSpec(shape, lambda i: tuple(0 for _ in shape))
    hn, on = pl.pallas_call(
        _gru_body,
        grid=(GRU_GRID,),
        in_specs=[
            pl.BlockSpec((1, BN, 128), lambda i: (i, 0, 0)),
            pl.BlockSpec((NC, 1, BN, 128), lambda i: (0, i, 0, 0)),
            pl.BlockSpec((1, BN, 128), lambda i: (i, 0, 0)),
            full((3, HP, HP)), full((3, HP, HP)), full((3, HP, HP)),
            full((3, 1, HP)), full((3, 1, HP)),
            full((HP, HP)), full((1, HP)),
        ],
        out_specs=[
            pl.BlockSpec((1, BN, 128), lambda i: (i, 0, 0)),
            pl.BlockSpec((1, BN, 128), lambda i: (i, 0, 0)),
        ],
        out_shape=[
            jax.ShapeDtypeStruct((GRU_GRID, BN, 128), jnp.float32),
            jax.ShapeDtypeStruct((GRU_GRID, BN, 128), jnp.float32),
        ],
    )(ni, parts, h, wn, wa, wh, bi, bh, wo, bo)
    return hn.reshape(N_ROWS, 128), on.reshape(N_ROWS, 128)


def kernel(node_inputs, src_ids, dst_ids, W1, b1, W2, b2, W3, b3,
           Wih, bih, Whh, bhh, Wout, bout):
    src_ids = src_ids.astype(jnp.int32)
    dst_ids = dst_ids.astype(jnp.int32)

    a1 = _pad2(W1[:, :NF].T, HP, 128)
    b1l = _pad2(W1[:, NF:].T, HP, 128)
    b1p = _pad2(b1[None, :], 1, 128)
    w2 = _pad2(W2.T, 128, 128)
    b2p = _pad2(b2[None, :], 1, 128)
    w3 = _pad2(W3.T, 128, HP)
    b3p = _pad2(b3[None, :], 1, HP)

    wn = jnp.stack([_pad2(Wih[g * NF:(g + 1) * NF, :NI].T, HP, HP)
                    for g in range(3)])
    wa = jnp.stack([_pad2(Wih[g * NF:(g + 1) * NF, NI:].T, HP, HP)
                    for g in range(3)])
    wh = jnp.stack([_pad2(Whh[g * NF:(g + 1) * NF, :].T, HP, HP)
                    for g in range(3)])
    bi = jnp.stack([_pad2(bih[None, g * NF:(g + 1) * NF], 1, HP)
                    for g in range(3)])
    bh = jnp.stack([_pad2(bhh[None, g * NF:(g + 1) * NF], 1, HP)
                    for g in range(3)])
    wo = _pad2(Wout.T, HP, HP)
    bo = _pad2(bout[None, :], 1, HP)

    ni = _pad2(node_inputs, N_NODES, HP).reshape(N_ROWS, 128)
    zeros_nodes = jnp.zeros((N_NODES, HP), jnp.float32)

    hidden = jnp.zeros((N_ROWS, 128), jnp.float32)
    outs = []
    for _ in range(N_ITERS):
        gs, gd = _sc_gather(hidden.reshape(N_NODES, HP), src_ids, dst_ids)
        msg = _tc_mlp(gs.reshape(E_ROWS, 128), gd.reshape(E_ROWS, 128),
                      a1, b1l, b1p, w2, b2p, w3, b3p)
        parts = _sc_scatter(msg.reshape(N_EDGES, HP), dst_ids, zeros_nodes)
        hidden, out_it = _tc_gru(ni, parts.reshape(NC, N_ROWS, 128), hidden,
                                 wn, wa, wh, bi, bh, wo, bo)
        outs.append(out_it)
    out = jnp.stack(outs, axis=0).reshape(N_ITERS, N_NODES, HP)
    return out[:, :, :NO]


# speedup vs baseline: 8.7740x; 1.0168x over previous
import functools

import jax
import jax.numpy as jnp
from jax import lax
from jax.experimental import pallas as pl
from jax.experimental.pallas import tpu as pltpu
from jax.experimental.pallas import tpu_sc as plsc

N_NODES = 100000
N_EDGES = 1600000
N_ITERS = 7
NF = 10
NI = 9
EF = 11
NO = 9

HP = 16
NC = 2
NS = 16
NW = NC * NS
E_PER_W = N_EDGES // NW
CHUNK = 2000
N_CHUNK = E_PER_W // CHUNK
SCHUNK = 1000
N_SCHUNK = E_PER_W // SCHUNK
NODES_PER_TILE = N_NODES // NS


def _pad2(a, r, c):
    return jnp.zeros((r, c), jnp.float32).at[: a.shape[0], : a.shape[1]].set(a)


def _sc_gather_body(table, src_hbm, dst_hbm, out_s, out_d,
                    idx_s, idx_d, rows_s, rows_d, sem_s, sem_d):
    wid = lax.axis_index("s") * NC + lax.axis_index("c")
    base0 = wid * E_PER_W

    @pl.loop(0, N_CHUNK)
    def _chunks(j):
        base = base0 + j * CHUNK
        pltpu.sync_copy(src_hbm.at[pl.ds(base, CHUNK)], idx_s)
        pltpu.sync_copy(dst_hbm.at[pl.ds(base, CHUNK)], idx_d)
        cp_s = pltpu.async_copy(table.at[idx_s], rows_s, sem_s)
        cp_d = pltpu.async_copy(table.at[idx_d], rows_d, sem_d)
        cp_s.wait()
        cp_d.wait()
        pltpu.sync_copy(rows_s, out_s.at[pl.ds(base, CHUNK)])
        pltpu.sync_copy(rows_d, out_d.at[pl.ds(base, CHUNK)])


@functools.cache
def _sc_gather_kernel():
    return pl.kernel(
        _sc_gather_body,
        out_type=(
            jax.ShapeDtypeStruct((N_EDGES, HP), jnp.float32),
            jax.ShapeDtypeStruct((N_EDGES, HP), jnp.float32),
        ),
        mesh=plsc.VectorSubcoreMesh(core_axis_name="c", subcore_axis_name="s",
                                    num_cores=NC, num_subcores=NS),
        scratch_types=[
            pltpu.VMEM((CHUNK,), jnp.int32),
            pltpu.VMEM((CHUNK,), jnp.int32),
            pltpu.VMEM((CHUNK, HP), jnp.float32),
            pltpu.VMEM((CHUNK, HP), jnp.float32),
            pltpu.SemaphoreType.DMA,
            pltpu.SemaphoreType.DMA,
        ],
        compiler_params=pltpu.CompilerParams(use_tc_tiling_on_sc=False),
    )


def _sc_gather(hidden, src_ids, dst_ids):
    return _sc_gather_kernel()(hidden, src_ids, dst_ids)


def _sc_scatter_body(msg_hbm, dst_hbm, zeros_hbm, out_part,
                     acc, idx_v, msg_v, sem):
    c = lax.axis_index("c")
    s = lax.axis_index("s")
    wid = s * NC + c
    nbase = s * NODES_PER_TILE
    pltpu.sync_copy(zeros_hbm.at[pl.ds(nbase, NODES_PER_TILE)],
                    acc.at[pl.ds(nbase, NODES_PER_TILE)])
    plsc.subcore_barrier()

    base0 = wid * E_PER_W

    @pl.loop(0, N_SCHUNK)
    def _chunks(j):
        base = base0 + j * SCHUNK
        pltpu.sync_copy(dst_hbm.at[pl.ds(base, SCHUNK)], idx_v)
        pltpu.sync_copy(msg_hbm.at[pl.ds(base, SCHUNK)], msg_v)
        pltpu.sync_copy(msg_v, acc.at[idx_v], add=True)

    plsc.subcore_barrier()
    pltpu.sync_copy(acc.at[pl.ds(nbase, NODES_PER_TILE)],
                    out_part.at[c, pl.ds(nbase, NODES_PER_TILE)])


@functools.cache
def _sc_scatter_kernel():
    return pl.kernel(
        _sc_scatter_body,
        out_type=jax.ShapeDtypeStruct((NC, N_NODES, HP), jnp.float32),
        mesh=plsc.VectorSubcoreMesh(core_axis_name="c", subcore_axis_name="s",
                                    num_cores=NC, num_subcores=NS),
        scratch_types=[
            pltpu.VMEM_SHARED((N_NODES, HP), jnp.float32),
            pltpu.VMEM((SCHUNK,), jnp.int32),
            pltpu.VMEM((SCHUNK, HP), jnp.float32),
            pltpu.SemaphoreType.DMA,
        ],
        compiler_params=pltpu.CompilerParams(use_tc_tiling_on_sc=False),
    )


def _sc_scatter(msg, dst_ids, zeros_nodes):
    return _sc_scatter_kernel()(msg, dst_ids, zeros_nodes)


PACK = 128 // HP
E_ROWS = N_EDGES // PACK
N_ROWS = N_NODES // PACK
BE = 800


def _mlp_body(gs_ref, gd_ref, w1_ref, b1_ref, w2_ref, b2_ref,
              w3_ref, b3_ref, out_ref):
    gs = gs_ref[...].astype(jnp.bfloat16)
    gd = gd_ref[...].astype(jnp.bfloat16)
    pieces = []
    for j in range(PACK):
        xs = gs[:, j * HP:(j + 1) * HP]
        xd = gd[:, j * HP:(j + 1) * HP]
        x = jnp.concatenate([xs, xd], axis=1)
        m1 = jnp.dot(x, w1_ref[...], preferred_element_type=jnp.float32)
        m1 = jnp.maximum(m1 + b1_ref[...], 0.0).astype(jnp.bfloat16)
        m2 = jnp.dot(m1, w2_ref[...], preferred_element_type=jnp.float32)
        m2 = jnp.maximum(m2 + b2_ref[...], 0.0).astype(jnp.bfloat16)
        m3 = jnp.dot(m2, w3_ref[...], preferred_element_type=jnp.float32)
        pieces.append(m3 + b3_ref[...])
    out_ref[...] = jnp.concatenate(pieces, axis=1)


def _tc_mlp(gs, gd, w1, b1, w2, b2, w3, b3):
    grid = (E_ROWS // BE,)
    full = lambda shape: pl.BlockSpec(shape, lambda i: (0, 0))
    return pl.pallas_call(
        _mlp_body,
        grid=grid,
        in_specs=[
            pl.BlockSpec((BE, 128), lambda i: (i, 0)),
            pl.BlockSpec((BE, 128), lambda i: (i, 0)),
            full((2 * HP, 128)), full((1, 128)),
            full((128, 128)), full((1, 128)),
            full((128, HP)), full((1, HP)),
        ],
        out_specs=pl.BlockSpec((BE, 128), lambda i: (i, 0)),
        out_shape=jax.ShapeDtypeStruct((E_ROWS, 128), jnp.float32),
    )(gs, gd, w1, b1, w2, b2, w3, b3)


GRU_GRID = 4
BN = N_ROWS // GRU_GRID


def _gru_body(ni_ref, part_ref, h_ref, wn_ref, wa_ref, wh_ref, bi_ref,
              bh_ref, wo_ref, bo_ref, hout_ref, oout_ref):
    ni = ni_ref[0]
    agg = part_ref[0, 0] + part_ref[1, 0]
    h = h_ref[0]
    h_pieces = []
    o_pieces = []
    for j in range(PACK):
        nij = ni[:, j * HP:(j + 1) * HP]
        aj = agg[:, j * HP:(j + 1) * HP]
        hj = h[:, j * HP:(j + 1) * HP]

        def gates(g):
            gx = jnp.dot(nij, wn_ref[g], preferred_element_type=jnp.float32)
            gx = gx + jnp.dot(aj, wa_ref[g],
                              preferred_element_type=jnp.float32)
            gx = gx + bi_ref[g]
            gh = jnp.dot(hj, wh_ref[g], preferred_element_type=jnp.float32)
            gh = gh + bh_ref[g]
            return gx, gh

        rx, rh = gates(0)
        zx, zh = gates(1)
        nx, nh = gates(2)
        r = jax.nn.sigmoid(rx + rh)
        z = jax.nn.sigmoid(zx + zh)
        n = jnp.tanh(nx + r * nh)
        hn = (1.0 - z) * n + z * hj
        h_pieces.append(hn)
        o_pieces.append(jnp.dot(hn, wo_ref[...],
                                preferred_element_type=jnp.float32)
                        + bo_ref[...])
    hout_ref[0] = jnp.concatenate(h_pieces, axis=1)
    oout_ref[0] = jnp.concatenate(o_pieces, axis=1)


def _tc_gru(ni, parts, h, wn, wa, wh, bi, bh, wo, bo):
    ni = ni.reshape(GRU_GRID, BN, 128)
    parts = parts.reshape(NC, GRU_GRID, BN, 128)
    h = h.reshape(GRU_GRID, BN, 128)
    full = lambda shape: pl.BlockSpec(shape, lambda i: tuple(0 for _ in shape))
    hn, on = pl.pallas_call(
        _gru_body,
        grid=(GRU_GRID,),
        in_specs=[
            pl.BlockSpec((1, BN, 128), lambda i: (i, 0, 0)),
            pl.BlockSpec((NC, 1, BN, 128), lambda i: (0, i, 0, 0)),
            pl.BlockSpec((1, BN, 128), lambda i: (i, 0, 0)),
            full((3, HP, HP)), full((3, HP, HP)), full((3, HP, HP)),
            full((3, 1, HP)), full((3, 1, HP)),
            full((HP, HP)), full((1, HP)),
        ],
        out_specs=[
            pl.BlockSpec((1, BN, 128), lambda i: (i, 0, 0)),
            pl.BlockSpec((1, BN, 128), lambda i: (i, 0, 0)),
        ],
        out_shape=[
            jax.ShapeDtypeStruct((GRU_GRID, BN, 128), jnp.float32),
            jax.ShapeDtypeStruct((GRU_GRID, BN, 128), jnp.float32),
        ],
    )(ni, parts, h, wn, wa, wh, bi, bh, wo, bo)
    return hn.reshape(N_ROWS, 128), on.reshape(N_ROWS, 128)


def kernel(node_inputs, src_ids, dst_ids, W1, b1, W2, b2, W3, b3,
           Wih, bih, Whh, bhh, Wout, bout):
    src_ids = src_ids.astype(jnp.int32)
    dst_ids = dst_ids.astype(jnp.int32)

    w1 = jnp.zeros((2 * HP, 128), jnp.float32)
    w1 = w1.at[:NF, :96].set(W1[:, :NF].T).at[HP:HP + NF, :96].set(W1[:, NF:].T)
    w1 = w1.astype(jnp.bfloat16)
    b1p = _pad2(b1[None, :], 1, 128)
    w2 = _pad2(W2.T, 128, 128).astype(jnp.bfloat16)
    b2p = _pad2(b2[None, :], 1, 128)
    w3 = _pad2(W3.T, 128, HP).astype(jnp.bfloat16)
    b3p = _pad2(b3[None, :], 1, HP)

    wn = jnp.stack([_pad2(Wih[g * NF:(g + 1) * NF, :NI].T, HP, HP)
                    for g in range(3)])
    wa = jnp.stack([_pad2(Wih[g * NF:(g + 1) * NF, NI:].T, HP, HP)
                    for g in range(3)])
    wh = jnp.stack([_pad2(Whh[g * NF:(g + 1) * NF, :].T, HP, HP)
                    for g in range(3)])
    bi = jnp.stack([_pad2(bih[None, g * NF:(g + 1) * NF], 1, HP)
                    for g in range(3)])
    bh = jnp.stack([_pad2(bhh[None, g * NF:(g + 1) * NF], 1, HP)
                    for g in range(3)])
    wo = _pad2(Wout.T, HP, HP)
    bo = _pad2(bout[None, :], 1, HP)

    ni = _pad2(node_inputs, N_NODES, HP).reshape(N_ROWS, 128)
    zeros_nodes = jnp.zeros((N_NODES, HP), jnp.float32)

    hidden = jnp.zeros((N_ROWS, 128), jnp.float32)
    outs = []
    for _ in range(N_ITERS):
        gs, gd = _sc_gather(hidden.reshape(N_NODES, HP), src_ids, dst_ids)
        msg = _tc_mlp(gs.reshape(E_ROWS, 128), gd.reshape(E_ROWS, 128),
                      w1, b1p, w2, b2p, w3, b3p)
        parts = _sc_scatter(msg.reshape(N_EDGES, HP), dst_ids, zeros_nodes)
        hidden, out_it = _tc_gru(ni, parts.reshape(NC, N_ROWS, 128), hidden,
                                 wn, wa, wh, bi, bh, wo, bo)
        outs.append(out_it)
    out = jnp.stack(outs, axis=0).reshape(N_ITERS, N_NODES, HP)
    return out[:, :, :NO]


# speedup vs baseline: 9.9940x; 1.1391x over previous
import functools

import jax
import jax.numpy as jnp
from jax import lax
from jax.experimental import pallas as pl
from jax.experimental.pallas import tpu as pltpu
from jax.experimental.pallas import tpu_sc as plsc

N_NODES = 100000
N_EDGES = 1600000
N_ITERS = 7
NF = 10
NI = 9
EF = 11
NO = 9

HP = 16
NC = 2
NS = 16
NW = NC * NS
E_PER_W = N_EDGES // NW
CHUNK = 2000
N_CHUNK = E_PER_W // CHUNK
SCHUNK = 1000
N_SCHUNK = E_PER_W // SCHUNK
NODES_PER_TILE = N_NODES // NS


def _pad2(a, r, c):
    return jnp.zeros((r, c), jnp.float32).at[: a.shape[0], : a.shape[1]].set(a)


def _sc_gather_body(table, src_hbm, dst_hbm, out_s, out_d,
                    idx_s, idx_d, rows_s, rows_d, sem_s, sem_d):
    wid = lax.axis_index("s") * NC + lax.axis_index("c")
    base0 = wid * E_PER_W

    @pl.loop(0, N_CHUNK)
    def _chunks(j):
        base = base0 + j * CHUNK
        pltpu.sync_copy(src_hbm.at[pl.ds(base, CHUNK)], idx_s)
        pltpu.sync_copy(dst_hbm.at[pl.ds(base, CHUNK)], idx_d)
        cp_s = pltpu.async_copy(table.at[idx_s], rows_s, sem_s)
        cp_d = pltpu.async_copy(table.at[idx_d], rows_d, sem_d)
        cp_s.wait()
        cp_d.wait()
        pltpu.sync_copy(rows_s, out_s.at[pl.ds(base, CHUNK)])
        pltpu.sync_copy(rows_d, out_d.at[pl.ds(base, CHUNK)])


@functools.cache
def _sc_gather_kernel():
    return pl.kernel(
        _sc_gather_body,
        out_type=(
            jax.ShapeDtypeStruct((N_EDGES, HP), jnp.float32),
            jax.ShapeDtypeStruct((N_EDGES, HP), jnp.float32),
        ),
        mesh=plsc.VectorSubcoreMesh(core_axis_name="c", subcore_axis_name="s",
                                    num_cores=NC, num_subcores=NS),
        scratch_types=[
            pltpu.VMEM((CHUNK,), jnp.int32),
            pltpu.VMEM((CHUNK,), jnp.int32),
            pltpu.VMEM((CHUNK, HP), jnp.float32),
            pltpu.VMEM((CHUNK, HP), jnp.float32),
            pltpu.SemaphoreType.DMA,
            pltpu.SemaphoreType.DMA,
        ],
        compiler_params=pltpu.CompilerParams(use_tc_tiling_on_sc=False),
    )


def _sc_gather(hidden, src_ids, dst_ids):
    return _sc_gather_kernel()(hidden, src_ids, dst_ids)


def _sc_scatter_body(msg_hbm, dst_hbm, zeros_hbm, out_part,
                     acc, idx_v, msg_v, sem):
    c = lax.axis_index("c")
    s = lax.axis_index("s")
    wid = s * NC + c
    nbase = s * NODES_PER_TILE
    pltpu.sync_copy(zeros_hbm.at[pl.ds(nbase, NODES_PER_TILE)],
                    acc.at[pl.ds(nbase, NODES_PER_TILE)])
    plsc.subcore_barrier()

    base0 = wid * E_PER_W

    @pl.loop(0, N_SCHUNK)
    def _chunks(j):
        base = base0 + j * SCHUNK
        pltpu.sync_copy(dst_hbm.at[pl.ds(base, SCHUNK)], idx_v)
        pltpu.sync_copy(msg_hbm.at[pl.ds(base, SCHUNK)], msg_v)
        pltpu.sync_copy(msg_v, acc.at[idx_v], add=True)

    plsc.subcore_barrier()
    pltpu.sync_copy(acc.at[pl.ds(nbase, NODES_PER_TILE)],
                    out_part.at[c, pl.ds(nbase, NODES_PER_TILE)])


@functools.cache
def _sc_scatter_kernel():
    return pl.kernel(
        _sc_scatter_body,
        out_type=jax.ShapeDtypeStruct((NC, N_NODES, HP), jnp.float32),
        mesh=plsc.VectorSubcoreMesh(core_axis_name="c", subcore_axis_name="s",
                                    num_cores=NC, num_subcores=NS),
        scratch_types=[
            pltpu.VMEM_SHARED((N_NODES, HP), jnp.float32),
            pltpu.VMEM((SCHUNK,), jnp.int32),
            pltpu.VMEM((SCHUNK, HP), jnp.float32),
            pltpu.SemaphoreType.DMA,
        ],
        compiler_params=pltpu.CompilerParams(use_tc_tiling_on_sc=False),
    )


def _sc_scatter(msg, dst_ids, zeros_nodes):
    return _sc_scatter_kernel()(msg, dst_ids, zeros_nodes)


PACK = 128 // HP
E_ROWS = N_EDGES // PACK
N_ROWS = N_NODES // PACK
BE = 800


def _mlp_body(gs_ref, gd_ref, w1_ref, b1_ref, w2_ref, b2_ref,
              w3_ref, b3_ref, out_ref):
    x = jnp.concatenate([gs_ref[...], gd_ref[...]],
                        axis=1).astype(jnp.bfloat16)
    acc = b3_ref[...]
    for j in range(PACK):
        m1 = jnp.dot(x, w1_ref[j], preferred_element_type=jnp.float32)
        m1 = jnp.maximum(m1 + b1_ref[...], 0.0).astype(jnp.bfloat16)
        m2 = jnp.dot(m1, w2_ref[...], preferred_element_type=jnp.float32)
        m2 = jnp.maximum(m2 + b2_ref[...], 0.0).astype(jnp.bfloat16)
        acc = acc + jnp.dot(m2, w3_ref[j],
                            preferred_element_type=jnp.float32)
    out_ref[...] = acc


def _tc_mlp(gs, gd, w1, b1, w2, b2, w3, b3):
    grid = (E_ROWS // BE,)
    full = lambda shape: pl.BlockSpec(shape, lambda i: tuple(0 for _ in shape))
    return pl.pallas_call(
        _mlp_body,
        grid=grid,
        in_specs=[
            pl.BlockSpec((BE, 128), lambda i: (i, 0)),
            pl.BlockSpec((BE, 128), lambda i: (i, 0)),
            full((PACK, 256, 128)), full((1, 128)),
            full((128, 128)), full((1, 128)),
            full((PACK, 128, 128)), full((1, 128)),
        ],
        out_specs=pl.BlockSpec((BE, 128), lambda i: (i, 0)),
        out_shape=jax.ShapeDtypeStruct((E_ROWS, 128), jnp.float32),
    )(gs, gd, w1, b1, w2, b2, w3, b3)


GRU_GRID = 4
BN = N_ROWS // GRU_GRID


def _gru_body(ni_ref, part_ref, h_ref, wn_ref, wa_ref, wh_ref, bi_ref,
              bh_ref, wo_ref, bo_ref, hout_ref, oout_ref):
    ni = ni_ref[0]
    agg = part_ref[0, 0] + part_ref[1, 0]
    h = h_ref[0]

    def gates(g):
        gx = jnp.dot(ni, wn_ref[g], preferred_element_type=jnp.float32)
        gx = gx + jnp.dot(agg, wa_ref[g], preferred_element_type=jnp.float32)
        gx = gx + bi_ref[g]
        gh = jnp.dot(h, wh_ref[g], preferred_element_type=jnp.float32)
        gh = gh + bh_ref[g]
        return gx, gh

    rx, rh = gates(0)
    zx, zh = gates(1)
    nx, nh = gates(2)
    r = jax.nn.sigmoid(rx + rh)
    z = jax.nn.sigmoid(zx + zh)
    n = jnp.tanh(nx + r * nh)
    hn = (1.0 - z) * n + z * h
    hout_ref[0] = hn
    oout_ref[0] = jnp.dot(hn, wo_ref[...],
                          preferred_element_type=jnp.float32) + bo_ref[...]


def _tc_gru(ni, parts, h, wn, wa, wh, bi, bh, wo, bo):
    ni = ni.reshape(GRU_GRID, BN, 128)
    parts = parts.reshape(NC, GRU_GRID, BN, 128)
    h = h.reshape(GRU_GRID, BN, 128)
    full = lambda shape: pl.BlockSpec(shape, lambda i: tuple(0 for _ in shape))
    hn, on = pl.pallas_call(
        _gru_body,
        grid=(GRU_GRID,),
        in_specs=[
            pl.BlockSpec((1, BN, 128), lambda i: (i, 0, 0)),
            pl.BlockSpec((NC, 1, BN, 128), lambda i: (0, i, 0, 0)),
            pl.BlockSpec((1, BN, 128), lambda i: (i, 0, 0)),
            full((3, 128, 128)), full((3, 128, 128)), full((3, 128, 128)),
            full((3, 1, 128)), full((3, 1, 128)),
            full((128, 128)), full((1, 128)),
        ],
        out_specs=[
            pl.BlockSpec((1, BN, 128), lambda i: (i, 0, 0)),
            pl.BlockSpec((1, BN, 128), lambda i: (i, 0, 0)),
        ],
        out_shape=[
            jax.ShapeDtypeStruct((GRU_GRID, BN, 128), jnp.float32),
            jax.ShapeDtypeStruct((GRU_GRID, BN, 128), jnp.float32),
        ],
    )(ni, parts, h, wn, wa, wh, bi, bh, wo, bo)
    return hn.reshape(N_ROWS, 128), on.reshape(N_ROWS, 128)


def kernel(node_inputs, src_ids, dst_ids, W1, b1, W2, b2, W3, b3,
           Wih, bih, Whh, bhh, Wout, bout):
    src_ids = src_ids.astype(jnp.int32)
    dst_ids = dst_ids.astype(jnp.int32)

    w1 = jnp.zeros((PACK, 256, 128), jnp.float32)
    w3 = jnp.zeros((PACK, 128, 128), jnp.float32)
    for j in range(PACK):
        w1 = w1.at[j, j * HP:j * HP + NF, :96].set(W1[:, :NF].T)
        w1 = w1.at[j, 128 + j * HP:128 + j * HP + NF, :96].set(W1[:, NF:].T)
        w3 = w3.at[j, :96, j * HP:j * HP + EF].set(W3.T)
    w1 = w1.astype(jnp.bfloat16)
    b1p = _pad2(b1[None, :], 1, 128)
    w2 = _pad2(W2.T, 128, 128).astype(jnp.bfloat16)
    b2p = _pad2(b2[None, :], 1, 128)
    w3 = w3.astype(jnp.bfloat16)
    b3p = jnp.tile(_pad2(b3[None, :], 1, HP), (1, PACK))

    eye8 = jnp.eye(PACK, dtype=jnp.float32)
    bd = lambda a: jnp.kron(eye8, _pad2(a, HP, HP))
    wn = jnp.stack([bd(Wih[g * NF:(g + 1) * NF, :NI].T) for g in range(3)])
    wa = jnp.stack([bd(Wih[g * NF:(g + 1) * NF, NI:].T) for g in range(3)])
    wh = jnp.stack([bd(Whh[g * NF:(g + 1) * NF, :].T) for g in range(3)])
    bi = jnp.stack([jnp.tile(_pad2(bih[None, g * NF:(g + 1) * NF], 1, HP),
                             (1, PACK)) for g in range(3)])
    bh = jnp.stack([jnp.tile(_pad2(bhh[None, g * NF:(g + 1) * NF], 1, HP),
                             (1, PACK)) for g in range(3)])
    wo = bd(Wout.T)
    bo = jnp.tile(_pad2(bout[None, :], 1, HP), (1, PACK))

    ni = _pad2(node_inputs, N_NODES, HP).reshape(N_ROWS, 128)
    zeros_nodes = jnp.zeros((N_NODES, HP), jnp.float32)

    hidden = jnp.zeros((N_ROWS, 128), jnp.float32)
    outs = []
    for _ in range(N_ITERS):
        gs, gd = _sc_gather(hidden.reshape(N_NODES, HP), src_ids, dst_ids)
        msg = _tc_mlp(gs.reshape(E_ROWS, 128), gd.reshape(E_ROWS, 128),
                      w1, b1p, w2, b2p, w3, b3p)
        parts = _sc_scatter(msg.reshape(N_EDGES, HP), dst_ids, zeros_nodes)
        hidden, out_it = _tc_gru(ni, parts.reshape(NC, N_ROWS, 128), hidden,
                                 wn, wa, wh, bi, bh, wo, bo)
        outs.append(out_it)
    out = jnp.stack(outs, axis=0).reshape(N_ITERS, N_NODES, HP)
    return out[:, :, :NO]


# speedup vs baseline: 11.1074x; 1.1114x over previous
import functools

import jax
import jax.numpy as jnp
from jax import lax
from jax.experimental import pallas as pl
from jax.experimental.pallas import tpu as pltpu
from jax.experimental.pallas import tpu_sc as plsc

N_NODES = 100000
N_EDGES = 1600000
N_ITERS = 7
NF = 10
NI = 9
EF = 11
NO = 9

HP = 16
NC = 2
NS = 16
NW = NC * NS
E_PER_W = N_EDGES // NW
CHUNK = 1000
N_CHUNK = E_PER_W // CHUNK
SCHUNK = 1000
N_SCHUNK = E_PER_W // SCHUNK
NODES_PER_TILE = N_NODES // NS


def _pad2(a, r, c):
    return jnp.zeros((r, c), jnp.float32).at[: a.shape[0], : a.shape[1]].set(a)


def _sc_gather_body(table, src_hbm, dst_hbm, out_s, out_d,
                    idx_s, idx_d, rows_s, rows_d, sem_g, sem_o):
    wid = lax.axis_index("s") * NC + lax.axis_index("c")
    base0 = wid * E_PER_W
    g_desc = [None] * N_CHUNK
    o_desc = [None] * N_CHUNK
    for j in range(N_CHUNK):
        b = j % 2
        base = base0 + j * CHUNK
        if j > 0:
            g_desc[j - 1][0].wait()
            g_desc[j - 1][1].wait()
            pbase = base0 + (j - 1) * CHUNK
            o_desc[j - 1] = (
                pltpu.async_copy(rows_s.at[1 - b], out_s.at[pl.ds(pbase, CHUNK)], sem_o),
                pltpu.async_copy(rows_d.at[1 - b], out_d.at[pl.ds(pbase, CHUNK)], sem_o),
            )
        pltpu.sync_copy(src_hbm.at[pl.ds(base, CHUNK)], idx_s.at[b])
        pltpu.sync_copy(dst_hbm.at[pl.ds(base, CHUNK)], idx_d.at[b])
        if j > 1:
            o_desc[j - 2][0].wait()
            o_desc[j - 2][1].wait()
        g_desc[j] = (
            pltpu.async_copy(table.at[idx_s.at[b]], rows_s.at[b], sem_g),
            pltpu.async_copy(table.at[idx_d.at[b]], rows_d.at[b], sem_g),
        )
    j = N_CHUNK - 1
    g_desc[j][0].wait()
    g_desc[j][1].wait()
    if N_CHUNK > 1:
        o_desc[j - 1][0].wait()
        o_desc[j - 1][1].wait()
    base = base0 + j * CHUNK
    pltpu.sync_copy(rows_s.at[j % 2], out_s.at[pl.ds(base, CHUNK)])
    pltpu.sync_copy(rows_d.at[j % 2], out_d.at[pl.ds(base, CHUNK)])


@functools.cache
def _sc_gather_kernel():
    return pl.kernel(
        _sc_gather_body,
        out_type=(
            jax.ShapeDtypeStruct((N_EDGES, HP), jnp.float32),
            jax.ShapeDtypeStruct((N_EDGES, HP), jnp.float32),
        ),
        mesh=plsc.VectorSubcoreMesh(core_axis_name="c", subcore_axis_name="s",
                                    num_cores=NC, num_subcores=NS),
        scratch_types=[
            pltpu.VMEM((2, CHUNK), jnp.int32),
            pltpu.VMEM((2, CHUNK), jnp.int32),
            pltpu.VMEM((2, CHUNK, HP), jnp.float32),
            pltpu.VMEM((2, CHUNK, HP), jnp.float32),
            pltpu.SemaphoreType.DMA,
            pltpu.SemaphoreType.DMA,
        ],
        compiler_params=pltpu.CompilerParams(use_tc_tiling_on_sc=False),
    )


def _sc_gather(hidden, src_ids, dst_ids):
    return _sc_gather_kernel()(hidden, src_ids, dst_ids)


def _sc_scatter_body(msg_hbm, dst_hbm, zeros_hbm, out_part,
                     acc, idx_v, msg_v, sem):
    c = lax.axis_index("c")
    s = lax.axis_index("s")
    wid = s * NC + c
    nbase = s * NODES_PER_TILE
    pltpu.sync_copy(zeros_hbm.at[pl.ds(nbase, NODES_PER_TILE)],
                    acc.at[pl.ds(nbase, NODES_PER_TILE)])
    plsc.subcore_barrier()

    base0 = wid * E_PER_W

    @pl.loop(0, N_SCHUNK)
    def _chunks(j):
        base = base0 + j * SCHUNK
        pltpu.sync_copy(dst_hbm.at[pl.ds(base, SCHUNK)], idx_v)
        pltpu.sync_copy(msg_hbm.at[pl.ds(base, SCHUNK)], msg_v)
        pltpu.sync_copy(msg_v, acc.at[idx_v], add=True)

    plsc.subcore_barrier()
    pltpu.sync_copy(acc.at[pl.ds(nbase, NODES_PER_TILE)],
                    out_part.at[c, pl.ds(nbase, NODES_PER_TILE)])


def _sc_degree_body(dst_hbm, ones_hbm, zeros_hbm, out_part,
                    acc, idx_v, ones_v, sem):
    c = lax.axis_index("c")
    s = lax.axis_index("s")
    wid = s * NC + c
    nbase = s * NODES_PER_TILE
    pltpu.sync_copy(zeros_hbm.at[pl.ds(nbase, NODES_PER_TILE)],
                    acc.at[pl.ds(nbase, NODES_PER_TILE)])
    pltpu.sync_copy(ones_hbm, ones_v)
    plsc.subcore_barrier()

    base0 = wid * E_PER_W
    pltpu.sync_copy(dst_hbm.at[pl.ds(base0, SCHUNK)], idx_v.at[0])
    for j in range(N_SCHUNK):
        b = j % 2
        if j + 1 < N_SCHUNK:
            d1 = pltpu.async_copy(
                dst_hbm.at[pl.ds(base0 + (j + 1) * SCHUNK, SCHUNK)],
                idx_v.at[1 - b], sem)
        pltpu.sync_copy(ones_v, acc.at[idx_v.at[b]], add=True)
        if j + 1 < N_SCHUNK:
            d1.wait()

    plsc.subcore_barrier()
    pltpu.sync_copy(acc.at[pl.ds(nbase, NODES_PER_TILE)],
                    out_part.at[c, pl.ds(nbase, NODES_PER_TILE)])


@functools.cache
def _sc_scatter_kernel():
    return pl.kernel(
        _sc_scatter_body,
        out_type=jax.ShapeDtypeStruct((NC, N_NODES, HP), jnp.float32),
        mesh=plsc.VectorSubcoreMesh(core_axis_name="c", subcore_axis_name="s",
                                    num_cores=NC, num_subcores=NS),
        scratch_types=[
            pltpu.VMEM_SHARED((N_NODES, HP), jnp.float32),
            pltpu.VMEM((SCHUNK,), jnp.int32),
            pltpu.VMEM((SCHUNK, HP), jnp.float32),
            pltpu.SemaphoreType.DMA,
        ],
        compiler_params=pltpu.CompilerParams(use_tc_tiling_on_sc=False),
    )


def _sc_scatter(msg, dst_ids, zeros_nodes):
    return _sc_scatter_kernel()(msg, dst_ids, zeros_nodes)


@functools.cache
def _sc_degree_kernel():
    return pl.kernel(
        _sc_degree_body,
        out_type=jax.ShapeDtypeStruct((NC, N_NODES, HP), jnp.float32),
        mesh=plsc.VectorSubcoreMesh(core_axis_name="c", subcore_axis_name="s",
                                    num_cores=NC, num_subcores=NS),
        scratch_types=[
            pltpu.VMEM_SHARED((N_NODES, HP), jnp.float32),
            pltpu.VMEM((2, SCHUNK), jnp.int32),
            pltpu.VMEM((SCHUNK, HP), jnp.float32),
            pltpu.SemaphoreType.DMA,
        ],
        compiler_params=pltpu.CompilerParams(use_tc_tiling_on_sc=False),
    )


def _sc_degree(dst_ids, ones_rows, zeros_nodes):
    return _sc_degree_kernel()(dst_ids, ones_rows, zeros_nodes)


PACK = 128 // HP
E_ROWS = N_EDGES // PACK
N_ROWS = N_NODES // PACK
BE = 800


def _mlp_body(gs_ref, gd_ref, w1_ref, b1_ref, w2_ref, b2_ref,
              w3_ref, b3_ref, out_ref):
    x = jnp.concatenate([gs_ref[...], gd_ref[...]],
                        axis=1).astype(jnp.bfloat16)
    acc = b3_ref[...]
    for j in range(PACK):
        m1 = jnp.dot(x, w1_ref[j], preferred_element_type=jnp.float32)
        m1 = jnp.maximum(m1 + b1_ref[...], 0.0).astype(jnp.bfloat16)
        m2 = jnp.dot(m1, w2_ref[...], preferred_element_type=jnp.float32)
        m2 = jnp.maximum(m2 + b2_ref[...], 0.0).astype(jnp.bfloat16)
        acc = acc + jnp.dot(m2, w3_ref[j],
                            preferred_element_type=jnp.float32)
    out_ref[...] = acc


def _tc_mlp(gs, gd, w1, b1, w2, b2, w3, b3):
    grid = (E_ROWS // BE,)
    full = lambda shape: pl.BlockSpec(shape, lambda i: tuple(0 for _ in shape))
    return pl.pallas_call(
        _mlp_body,
        grid=grid,
        in_specs=[
            pl.BlockSpec((BE, 128), lambda i: (i, 0)),
            pl.BlockSpec((BE, 128), lambda i: (i, 0)),
            full((PACK, 256, 128)), full((1, 128)),
            full((128, 128)), full((1, 128)),
            full((PACK, 128, 128)), full((1, 128)),
        ],
        out_specs=pl.BlockSpec((BE, 128), lambda i: (i, 0)),
        out_shape=jax.ShapeDtypeStruct((E_ROWS, 128), jnp.float32),
    )(gs, gd, w1, b1, w2, b2, w3, b3)


def _tc_mlp_small(gs, gd, w1, b1, w2, b2, w3, b3):
    full = lambda shape: pl.BlockSpec(shape, lambda: tuple(0 for _ in shape))
    return pl.pallas_call(
        _mlp_body,
        in_specs=[
            full((8, 128)), full((8, 128)),
            full((PACK, 256, 128)), full((1, 128)),
            full((128, 128)), full((1, 128)),
            full((PACK, 128, 128)), full((1, 128)),
        ],
        out_specs=full((8, 128)),
        out_shape=jax.ShapeDtypeStruct((8, 128), jnp.float32),
    )(gs, gd, w1, b1, w2, b2, w3, b3)


GRU_GRID = 4
BN = N_ROWS // GRU_GRID


def _gru_body(ni_ref, part_ref, h_ref, sc_ref, wn_ref, wa_ref, wh_ref,
              bi_ref, bh_ref, wo_ref, bo_ref, hout_ref, oout_ref):
    ni = ni_ref[0]
    agg = (part_ref[0, 0] + part_ref[1, 0]) * sc_ref[...]
    h = h_ref[0]

    def gates(g):
        gx = jnp.dot(ni, wn_ref[g], preferred_element_type=jnp.float32)
        gx = gx + jnp.dot(agg, wa_ref[g], preferred_element_type=jnp.float32)
        gx = gx + bi_ref[g]
        gh = jnp.dot(h, wh_ref[g], preferred_element_type=jnp.float32)
        gh = gh + bh_ref[g]
        return gx, gh

    rx, rh = gates(0)
    zx, zh = gates(1)
    nx, nh = gates(2)
    r = jax.nn.sigmoid(rx + rh)
    z = jax.nn.sigmoid(zx + zh)
    n = jnp.tanh(nx + r * nh)
    hn = (1.0 - z) * n + z * h
    hout_ref[0] = hn
    oout_ref[0] = jnp.dot(hn, wo_ref[...],
                          preferred_element_type=jnp.float32) + bo_ref[...]


def _tc_gru(ni, parts, h, sc, wn, wa, wh, bi, bh, wo, bo):
    ni = ni.reshape(GRU_GRID, BN, 128)
    parts = parts.reshape(NC, GRU_GRID, BN, 128)
    h = h.reshape(GRU_GRID, BN, 128)
    full = lambda shape: pl.BlockSpec(shape, lambda i: tuple(0 for _ in shape))
    hn, on = pl.pallas_call(
        _gru_body,
        grid=(GRU_GRID,),
        in_specs=[
            pl.BlockSpec((1, BN, 128), lambda i: (i, 0, 0)),
            pl.BlockSpec((NC, 1, BN, 128), lambda i: (0, i, 0, 0)),
            pl.BlockSpec((1, BN, 128), lambda i: (i, 0, 0)),
            full((1, 128)),
            full((3, 128, 128)), full((3, 128, 128)), full((3, 128, 128)),
            full((3, 1, 128)), full((3, 1, 128)),
            full((128, 128)), full((1, 128)),
        ],
        out_specs=[
            pl.BlockSpec((1, BN, 128), lambda i: (i, 0, 0)),
            pl.BlockSpec((1, BN, 128), lambda i: (i, 0, 0)),
        ],
        out_shape=[
            jax.ShapeDtypeStruct((GRU_GRID, BN, 128), jnp.float32),
            jax.ShapeDtypeStruct((GRU_GRID, BN, 128), jnp.float32),
        ],
    )(ni, parts, h, sc, wn, wa, wh, bi, bh, wo, bo)
    return hn.reshape(N_ROWS, 128), on.reshape(N_ROWS, 128)


def kernel(node_inputs, src_ids, dst_ids, W1, b1, W2, b2, W3, b3,
           Wih, bih, Whh, bhh, Wout, bout):
    src_ids = src_ids.astype(jnp.int32)
    dst_ids = dst_ids.astype(jnp.int32)

    w1 = jnp.zeros((PACK, 256, 128), jnp.float32)
    w3 = jnp.zeros((PACK, 128, 128), jnp.float32)
    for j in range(PACK):
        w1 = w1.at[j, j * HP:j * HP + NF, :96].set(W1[:, :NF].T)
        w1 = w1.at[j, 128 + j * HP:128 + j * HP + NF, :96].set(W1[:, NF:].T)
        w3 = w3.at[j, :96, j * HP:j * HP + EF].set(W3.T)
    w1 = w1.astype(jnp.bfloat16)
    b1p = _pad2(b1[None, :], 1, 128)
    w2 = _pad2(W2.T, 128, 128).astype(jnp.bfloat16)
    b2p = _pad2(b2[None, :], 1, 128)
    w3 = w3.astype(jnp.bfloat16)
    b3p = jnp.tile(_pad2(b3[None, :], 1, HP), (1, PACK))

    eye8 = jnp.eye(PACK, dtype=jnp.float32)
    bd = lambda a: jnp.kron(eye8, _pad2(a, HP, HP))
    wn = jnp.stack([bd(Wih[g * NF:(g + 1) * NF, :NI].T) for g in range(3)])
    wa = jnp.stack([bd(Wih[g * NF:(g + 1) * NF, NI:].T) for g in range(3)])
    wh = jnp.stack([bd(Whh[g * NF:(g + 1) * NF, :].T) for g in range(3)])
    bi = jnp.stack([jnp.tile(_pad2(bih[None, g * NF:(g + 1) * NF], 1, HP),
                             (1, PACK)) for g in range(3)])
    bh = jnp.stack([jnp.tile(_pad2(bhh[None, g * NF:(g + 1) * NF], 1, HP),
                             (1, PACK)) for g in range(3)])
    wo = bd(Wout.T)
    bo = jnp.tile(_pad2(bout[None, :], 1, HP), (1, PACK))

    ni = _pad2(node_inputs, N_NODES, HP).reshape(N_ROWS, 128)
    zeros_nodes = jnp.zeros((N_NODES, HP), jnp.float32)

    ones_sc = jnp.ones((1, 128), jnp.float32)
    ones_rows = jnp.ones((SCHUNK, HP), jnp.float32)

    deg_parts = _sc_degree(dst_ids, ones_rows, zeros_nodes)
    z8 = jnp.zeros((8, 128), jnp.float32)
    msg0 = _tc_mlp_small(z8, z8, w1, b1p, w2, b2p, w3, b3p)[0:1, :]

    hidden = jnp.zeros((N_ROWS, 128), jnp.float32)
    outs = []
    for it in range(N_ITERS):
        if it == 0:
            parts, sc = deg_parts, msg0
        else:
            gs, gd = _sc_gather(hidden.reshape(N_NODES, HP), src_ids,
                                dst_ids)
            msg = _tc_mlp(gs.reshape(E_ROWS, 128), gd.reshape(E_ROWS, 128),
                          w1, b1p, w2, b2p, w3, b3p)
            parts = _sc_scatter(msg.reshape(N_EDGES, HP), dst_ids,
                                zeros_nodes)
            sc = ones_sc
        hidden, out_it = _tc_gru(ni, parts.reshape(NC, N_ROWS, 128), hidden,
                                 sc, wn, wa, wh, bi, bh, wo, bo)
        outs.append(out_it)
    out = jnp.stack(outs, axis=0).reshape(N_ITERS, N_NODES, HP)
    return out[:, :, :NO]


# speedup vs baseline: 12.3799x; 1.1146x over previous
import functools

import jax
import jax.numpy as jnp
from jax import lax
from jax.experimental import pallas as pl
from jax.experimental.pallas import tpu as pltpu
from jax.experimental.pallas import tpu_sc as plsc

N_NODES = 100000
N_EDGES = 1600000
N_ITERS = 7
NF = 10
NI = 9
EF = 11
NO = 9

HP = 16
NC = 2
NS = 16
NW = NC * NS
E_PER_W = N_EDGES // NW
CHUNK = 2000
N_CHUNK = E_PER_W // CHUNK
SCHUNK = 1000
N_SCHUNK = E_PER_W // SCHUNK
NODES_PER_TILE = N_NODES // NS


def _pad2(a, r, c):
    return jnp.zeros((r, c), jnp.float32).at[: a.shape[0], : a.shape[1]].set(a)


def _sc_gather_body(table, src_hbm, dst_hbm, out_s, out_d,
                    idx_s, idx_d, rows_s, rows_d, sem_g, sem_o):
    wid = lax.axis_index("s") * NC + lax.axis_index("c")
    base0 = wid * E_PER_W

    @pl.loop(0, N_CHUNK)
    def _chunks(j):
        base = base0 + j * CHUNK
        pltpu.sync_copy(src_hbm.at[pl.ds(base, CHUNK)], idx_s)
        pltpu.sync_copy(dst_hbm.at[pl.ds(base, CHUNK)], idx_d)
        cp_s = pltpu.async_copy(table.at[idx_s], rows_s, sem_g)
        cp_d = pltpu.async_copy(table.at[idx_d], rows_d, sem_o)
        cp_s.wait()
        cp_d.wait()
        pltpu.sync_copy(rows_s, out_s.at[pl.ds(base, CHUNK)])
        pltpu.sync_copy(rows_d, out_d.at[pl.ds(base, CHUNK)])


@functools.cache
def _sc_gather_kernel():
    return pl.kernel(
        _sc_gather_body,
        out_type=(
            jax.ShapeDtypeStruct((N_EDGES, HP), jnp.float32),
            jax.ShapeDtypeStruct((N_EDGES, HP), jnp.float32),
        ),
        mesh=plsc.VectorSubcoreMesh(core_axis_name="c", subcore_axis_name="s",
                                    num_cores=NC, num_subcores=NS),
        scratch_types=[
            pltpu.VMEM((CHUNK,), jnp.int32),
            pltpu.VMEM((CHUNK,), jnp.int32),
            pltpu.VMEM((CHUNK, HP), jnp.float32),
            pltpu.VMEM((CHUNK, HP), jnp.float32),
            pltpu.SemaphoreType.DMA,
            pltpu.SemaphoreType.DMA,
        ],
        compiler_params=pltpu.CompilerParams(use_tc_tiling_on_sc=False),
    )


def _sc_gather(hidden, src_ids, dst_ids):
    return _sc_gather_kernel()(hidden, src_ids, dst_ids)


def _sc_scatter_body(msg_hbm, dst_hbm, zeros_hbm, out_part,
                     acc, idx_v, msg_v, sem):
    c = lax.axis_index("c")
    s = lax.axis_index("s")
    wid = s * NC + c
    nbase = s * NODES_PER_TILE
    pltpu.sync_copy(zeros_hbm.at[pl.ds(nbase, NODES_PER_TILE)],
                    acc.at[pl.ds(nbase, NODES_PER_TILE)])
    plsc.subcore_barrier()

    base0 = wid * E_PER_W

    @pl.loop(0, N_SCHUNK)
    def _chunks(j):
        base = base0 + j * SCHUNK
        pltpu.sync_copy(dst_hbm.at[pl.ds(base, SCHUNK)], idx_v)
        pltpu.sync_copy(msg_hbm.at[pl.ds(base, SCHUNK)], msg_v)
        pltpu.sync_copy(msg_v, acc.at[idx_v], add=True)

    plsc.subcore_barrier()
    pltpu.sync_copy(acc.at[pl.ds(nbase, NODES_PER_TILE)],
                    out_part.at[c, pl.ds(nbase, NODES_PER_TILE)])


def _sc_degree_body(dst_hbm, ones_hbm, zeros_hbm, out_part,
                    acc, idx_v, ones_v, sem):
    c = lax.axis_index("c")
    s = lax.axis_index("s")
    wid = s * NC + c
    nbase = s * NODES_PER_TILE
    pltpu.sync_copy(zeros_hbm.at[pl.ds(nbase, NODES_PER_TILE)],
                    acc.at[pl.ds(nbase, NODES_PER_TILE)])
    pltpu.sync_copy(ones_hbm, ones_v)
    plsc.subcore_barrier()

    base0 = wid * E_PER_W
    pltpu.sync_copy(dst_hbm.at[pl.ds(base0, SCHUNK)], idx_v.at[0])
    for j in range(N_SCHUNK):
        b = j % 2
        if j + 1 < N_SCHUNK:
            d1 = pltpu.async_copy(
                dst_hbm.at[pl.ds(base0 + (j + 1) * SCHUNK, SCHUNK)],
                idx_v.at[1 - b], sem)
        pltpu.sync_copy(ones_v, acc.at[idx_v.at[b]], add=True)
        if j + 1 < N_SCHUNK:
            d1.wait()

    plsc.subcore_barrier()
    pltpu.sync_copy(acc.at[pl.ds(nbase, NODES_PER_TILE)],
                    out_part.at[c, pl.ds(nbase, NODES_PER_TILE)])


@functools.cache
def _sc_scatter_kernel():
    return pl.kernel(
        _sc_scatter_body,
        out_type=jax.ShapeDtypeStruct((NC, N_NODES, HP), jnp.float32),
        mesh=plsc.VectorSubcoreMesh(core_axis_name="c", subcore_axis_name="s",
                                    num_cores=NC, num_subcores=NS),
        scratch_types=[
            pltpu.VMEM_SHARED((N_NODES, HP), jnp.float32),
            pltpu.VMEM((SCHUNK,), jnp.int32),
            pltpu.VMEM((SCHUNK, HP), jnp.float32),
            pltpu.SemaphoreType.DMA,
        ],
        compiler_params=pltpu.CompilerParams(use_tc_tiling_on_sc=False),
    )


def _sc_scatter(msg, dst_ids, zeros_nodes):
    return _sc_scatter_kernel()(msg, dst_ids, zeros_nodes)


@functools.cache
def _sc_degree_kernel():
    return pl.kernel(
        _sc_degree_body,
        out_type=jax.ShapeDtypeStruct((NC, N_NODES, HP), jnp.float32),
        mesh=plsc.VectorSubcoreMesh(core_axis_name="c", subcore_axis_name="s",
                                    num_cores=NC, num_subcores=NS),
        scratch_types=[
            pltpu.VMEM_SHARED((N_NODES, HP), jnp.float32),
            pltpu.VMEM((2, SCHUNK), jnp.int32),
            pltpu.VMEM((SCHUNK, HP), jnp.float32),
            pltpu.SemaphoreType.DMA,
        ],
        compiler_params=pltpu.CompilerParams(use_tc_tiling_on_sc=False),
    )


def _sc_degree(dst_ids, ones_rows, zeros_nodes):
    return _sc_degree_kernel()(dst_ids, ones_rows, zeros_nodes)


PACK = 128 // HP
E_ROWS = N_EDGES // PACK
N_ROWS = N_NODES // PACK
BE = 1600


def _mlp_body(gs_ref, gd_ref, w1_ref, b1_ref, w2_ref, b2_ref,
              w3_ref, b3_ref, out_ref):
    x = jnp.concatenate([gs_ref[...], gd_ref[...]],
                        axis=1).astype(jnp.bfloat16)
    acc = b3_ref[...]
    for j in range(PACK):
        m1 = jnp.dot(x, w1_ref[j], preferred_element_type=jnp.float32)
        m1 = jnp.maximum(m1 + b1_ref[...], 0.0).astype(jnp.bfloat16)
        m2 = jnp.dot(m1, w2_ref[...], preferred_element_type=jnp.float32)
        m2 = jnp.maximum(m2 + b2_ref[...], 0.0).astype(jnp.bfloat16)
        acc = acc + jnp.dot(m2, w3_ref[j],
                            preferred_element_type=jnp.float32)
    out_ref[...] = acc


def _tc_mlp(gs, gd, w1, b1, w2, b2, w3, b3):
    grid = (E_ROWS // BE,)
    full = lambda shape: pl.BlockSpec(shape, lambda i: tuple(0 for _ in shape))
    return pl.pallas_call(
        _mlp_body,
        grid=grid,
        in_specs=[
            pl.BlockSpec((BE, 128), lambda i: (i, 0)),
            pl.BlockSpec((BE, 128), lambda i: (i, 0)),
            full((PACK, 256, 128)), full((1, 128)),
            full((128, 128)), full((1, 128)),
            full((PACK, 128, 128)), full((1, 128)),
        ],
        out_specs=pl.BlockSpec((BE, 128), lambda i: (i, 0)),
        out_shape=jax.ShapeDtypeStruct((E_ROWS, 128), jnp.float32),
    )(gs, gd, w1, b1, w2, b2, w3, b3)


def _tc_mlp_small(gs, gd, w1, b1, w2, b2, w3, b3):
    full = lambda shape: pl.BlockSpec(shape, lambda: tuple(0 for _ in shape))
    return pl.pallas_call(
        _mlp_body,
        in_specs=[
            full((8, 128)), full((8, 128)),
            full((PACK, 256, 128)), full((1, 128)),
            full((128, 128)), full((1, 128)),
            full((PACK, 128, 128)), full((1, 128)),
        ],
        out_specs=full((8, 128)),
        out_shape=jax.ShapeDtypeStruct((8, 128), jnp.float32),
    )(gs, gd, w1, b1, w2, b2, w3, b3)


GRU_GRID = 4
BN = N_ROWS // GRU_GRID


def _gru_body(ni_ref, part_ref, h_ref, sc_ref, wn_ref, wa_ref, wh_ref,
              bi_ref, bh_ref, wo_ref, bo_ref, hout_ref, oout_ref):
    ni = ni_ref[0]
    agg = (part_ref[0, 0] + part_ref[1, 0]) * sc_ref[...]
    h = h_ref[0]

    def gates(g):
        gx = jnp.dot(ni, wn_ref[g], preferred_element_type=jnp.float32)
        gx = gx + jnp.dot(agg, wa_ref[g], preferred_element_type=jnp.float32)
        gx = gx + bi_ref[g]
        gh = jnp.dot(h, wh_ref[g], preferred_element_type=jnp.float32)
        gh = gh + bh_ref[g]
        return gx, gh

    rx, rh = gates(0)
    zx, zh = gates(1)
    nx, nh = gates(2)
    r = jax.nn.sigmoid(rx + rh)
    z = jax.nn.sigmoid(zx + zh)
    n = jnp.tanh(nx + r * nh)
    hn = (1.0 - z) * n + z * h
    hout_ref[0] = hn
    oout_ref[0] = jnp.dot(hn, wo_ref[...],
                          preferred_element_type=jnp.float32) + bo_ref[...]


def _tc_gru(ni, parts, h, sc, wn, wa, wh, bi, bh, wo, bo):
    ni = ni.reshape(GRU_GRID, BN, 128)
    parts = parts.reshape(NC, GRU_GRID, BN, 128)
    h = h.reshape(GRU_GRID, BN, 128)
    full = lambda shape: pl.BlockSpec(shape, lambda i: tuple(0 for _ in shape))
    hn, on = pl.pallas_call(
        _gru_body,
        grid=(GRU_GRID,),
        in_specs=[
            pl.BlockSpec((1, BN, 128), lambda i: (i, 0, 0)),
            pl.BlockSpec((NC, 1, BN, 128), lambda i: (0, i, 0, 0)),
            pl.BlockSpec((1, BN, 128), lambda i: (i, 0, 0)),
            full((1, 128)),
            full((3, 128, 128)), full((3, 128, 128)), full((3, 128, 128)),
            full((3, 1, 128)), full((3, 1, 128)),
            full((128, 128)), full((1, 128)),
        ],
        out_specs=[
            pl.BlockSpec((1, BN, 128), lambda i: (i, 0, 0)),
            pl.BlockSpec((1, BN, 128), lambda i: (i, 0, 0)),
        ],
        out_shape=[
            jax.ShapeDtypeStruct((GRU_GRID, BN, 128), jnp.float32),
            jax.ShapeDtypeStruct((GRU_GRID, BN, 128), jnp.float32),
        ],
    )(ni, parts, h, sc, wn, wa, wh, bi, bh, wo, bo)
    return hn.reshape(N_ROWS, 128), on.reshape(N_ROWS, 128)


def kernel(node_inputs, src_ids, dst_ids, W1, b1, W2, b2, W3, b3,
           Wih, bih, Whh, bhh, Wout, bout):
    src_ids = src_ids.astype(jnp.int32)
    dst_ids = dst_ids.astype(jnp.int32)

    w1 = jnp.zeros((PACK, 256, 128), jnp.float32)
    w3 = jnp.zeros((PACK, 128, 128), jnp.float32)
    for j in range(PACK):
        w1 = w1.at[j, j * HP:j * HP + NF, :96].set(W1[:, :NF].T)
        w1 = w1.at[j, 128 + j * HP:128 + j * HP + NF, :96].set(W1[:, NF:].T)
        w3 = w3.at[j, :96, j * HP:j * HP + EF].set(W3.T)
    w1 = w1.astype(jnp.bfloat16)
    b1p = _pad2(b1[None, :], 1, 128)
    w2 = _pad2(W2.T, 128, 128).astype(jnp.bfloat16)
    b2p = _pad2(b2[None, :], 1, 128)
    w3 = w3.astype(jnp.bfloat16)
    b3p = jnp.tile(_pad2(b3[None, :], 1, HP), (1, PACK))

    eye8 = jnp.eye(PACK, dtype=jnp.float32)
    bd = lambda a: jnp.kron(eye8, _pad2(a, HP, HP))
    wn = jnp.stack([bd(Wih[g * NF:(g + 1) * NF, :NI].T) for g in range(3)])
    wa = jnp.stack([bd(Wih[g * NF:(g + 1) * NF, NI:].T) for g in range(3)])
    wh = jnp.stack([bd(Whh[g * NF:(g + 1) * NF, :].T) for g in range(3)])
    bi = jnp.stack([jnp.tile(_pad2(bih[None, g * NF:(g + 1) * NF], 1, HP),
                             (1, PACK)) for g in range(3)])
    bh = jnp.stack([jnp.tile(_pad2(bhh[None, g * NF:(g + 1) * NF], 1, HP),
                             (1, PACK)) for g in range(3)])
    wo = bd(Wout.T)
    bo = jnp.tile(_pad2(bout[None, :], 1, HP), (1, PACK))

    ni = _pad2(node_inputs, N_NODES, HP).reshape(N_ROWS, 128)
    zeros_nodes = jnp.zeros((N_NODES, HP), jnp.float32)

    ones_sc = jnp.ones((1, 128), jnp.float32)
    ones_rows = jnp.ones((SCHUNK, HP), jnp.float32)

    deg_parts = _sc_degree(dst_ids, ones_rows, zeros_nodes)
    z8 = jnp.zeros((8, 128), jnp.float32)
    msg0 = _tc_mlp_small(z8, z8, w1, b1p, w2, b2p, w3, b3p)[0:1, :]

    hidden = jnp.zeros((N_ROWS, 128), jnp.float32)
    outs = []
    for it in range(N_ITERS):
        if it == 0:
            parts, sc = deg_parts, msg0
        else:
            gs, gd = _sc_gather(hidden.reshape(N_NODES, HP), src_ids,
                                dst_ids)
            msg = _tc_mlp(gs.reshape(E_ROWS, 128), gd.reshape(E_ROWS, 128),
                          w1, b1p, w2, b2p, w3, b3p)
            parts = _sc_scatter(msg.reshape(N_EDGES, HP), dst_ids,
                                zeros_nodes)
            sc = ones_sc
        hidden, out_it = _tc_gru(ni, parts.reshape(NC, N_ROWS, 128), hidden,
                                 sc, wn, wa, wh, bi, bh, wo, bo)
        outs.append(out_it)
    out = jnp.stack(outs, axis=0).reshape(N_ITERS, N_NODES, HP)
    return out[:, :, :NO]
